# baseline (speedup 1.0000x reference)
import functools

import jax
import jax.numpy as jnp
from jax import lax
from jax.experimental import pallas as pl
from jax.experimental.pallas import tpu as pltpu
from jax.experimental.pallas import tpu_sc as plsc

Q = 1024
D = 128
KB = 4096
W = 1024
U = KB // W


def _argmin_body(q_ref, k_ref, lab_ref, kt_ref, labt_ref, idx_ref,
                 qn_ref, rmin_ref, rjdx_ref):
    j = pl.program_id(0)
    nblk = pl.num_programs(0)

    @pl.when(j == 0)
    def _init():
        q = q_ref[...]
        qn_ref[...] = (
            q / (jnp.sqrt(jnp.sum(q * q, axis=1, keepdims=True)) + 1e-12)
        ).astype(jnp.bfloat16)
        rmin_ref[...] = jnp.full_like(rmin_ref, jnp.inf)
        rjdx_ref[...] = jnp.zeros_like(rjdx_ref)

    k = k_ref[...]
    kn = (k / (jnp.sqrt(jnp.sum(k * k, axis=1, keepdims=True)) + 1e-12)
          ).astype(jnp.bfloat16)

    s = lax.dot_general(
        qn_ref[...], kn,
        dimension_numbers=(((1,), (1,)), ((), ())),
        preferred_element_type=jnp.float32,
    )
    lab = lab_ref[...].reshape(1, KB)
    pen = jnp.where(lab > 0, jnp.float32(1.0), jnp.float32(jnp.inf))
    dd = pen - s

    b = j * U
    vals = [dd[:, h * W:(h + 1) * W] for h in range(U)]
    idxs = [b + h for h in range(U)]
    while len(vals) > 1:
        nv, ni = [], []
        for p in range(0, len(vals) - 1, 2):
            cl = vals[p] <= vals[p + 1]
            nv.append(jnp.where(cl, vals[p], vals[p + 1]))
            ni.append(jnp.where(cl, idxs[p], idxs[p + 1]))
        if len(vals) % 2:
            nv.append(vals[-1])
            ni.append(idxs[-1])
        vals, idxs = nv, ni
    mloc, jloc = vals[0], idxs[0]

    upd = mloc < rmin_ref[...]
    rjdx_ref[...] = jnp.where(upd, jloc, rjdx_ref[...])
    rmin_ref[...] = jnp.where(upd, mloc, rmin_ref[...])

    @pl.when(j == nblk - 1)
    def _final():
        nmain = nblk * KB
        kt = kt_ref[...]
        knt = (kt / (jnp.sqrt(jnp.sum(kt * kt, axis=1, keepdims=True))
                     + 1e-12)).astype(jnp.bfloat16)
        st = lax.dot_general(
            qn_ref[...], knt,
            dimension_numbers=(((1,), (1,)), ((), ())),
            preferred_element_type=jnp.float32,
        )
        labt = labt_ref[...]
        pent = jnp.where(labt > 0, jnp.float32(1.0), jnp.float32(jnp.inf))
        ddt = pent - st
        tmin = jnp.min(ddt, axis=1, keepdims=True)
        targ = (jnp.argmin(ddt, axis=1).astype(jnp.int32).reshape(Q, 1)
                + nmain)

        rmin = rmin_ref[...]
        m = jnp.min(rmin, axis=1, keepdims=True)
        c = lax.broadcasted_iota(jnp.int32, (Q, W), 1)
        cand = jnp.where(rmin == m, rjdx_ref[...] * W + c,
                         jnp.int32(2**31 - 1))
        idx_main = jnp.min(cand, axis=1, keepdims=True)
        idx_ref[...] = jnp.where(tmin < m, targ, idx_main)


def _nearest_index(queries, keys, label_confi):
    K = keys.shape[0]
    nblk = K // KB
    nmain = nblk * KB
    kt = K - nmain
    lab = label_confi.astype(jnp.int32)
    lab3d = lab[:nmain].reshape(nblk, 1, KB)
    keys_tail = keys[nmain:]
    lab_tail = lab[nmain:].reshape(1, kt)
    idx = pl.pallas_call(
        _argmin_body,
        grid=(nblk,),
        in_specs=[
            pl.BlockSpec((Q, D), lambda j: (0, 0)),
            pl.BlockSpec((KB, D), lambda j: (j, 0)),
            pl.BlockSpec((1, 1, KB), lambda j: (j, 0, 0)),
            pl.BlockSpec((kt, D), lambda j: (0, 0)),
            pl.BlockSpec((1, kt), lambda j: (0, 0)),
        ],
        out_specs=pl.BlockSpec((Q, 1), lambda j: (0, 0)),
        out_shape=jax.ShapeDtypeStruct((Q, 1), jnp.int32),
        scratch_shapes=[
            pltpu.VMEM((Q, D), jnp.bfloat16),
            pltpu.VMEM((Q, W), jnp.float32),
            pltpu.VMEM((Q, W), jnp.int32),
        ],
    )(queries, keys, lab3d, keys_tail, lab_tail)
    return idx.reshape(Q)


def _make_sc_gather(V, B, Dm):
    NC, NS = 2, 16
    NW = NC * NS
    b_per_w = B // NW
    mesh = plsc.VectorSubcoreMesh(core_axis_name="c", subcore_axis_name="s")

    @functools.partial(
        pl.kernel,
        mesh=mesh,
        out_type=jax.ShapeDtypeStruct((B, Dm), jnp.float32),
        scratch_types=[
            pltpu.VMEM((b_per_w,), jnp.int32),
            pltpu.VMEM((b_per_w, Dm), jnp.float32),
            pltpu.SemaphoreType.DMA,
        ],
    )
    def gather_rows(idx_hbm, table_hbm, out_hbm, idx_v, rows_v, sem):
        wid = lax.axis_index("s") * NC + lax.axis_index("c")
        base = wid * b_per_w
        pltpu.sync_copy(idx_hbm.at[pl.ds(base, b_per_w)], idx_v)
        pltpu.async_copy(table_hbm.at[idx_v], rows_v, sem).wait()
        pltpu.sync_copy(rows_v, out_hbm.at[pl.ds(base, b_per_w)])

    return gather_rows


def kernel(queries, keys, label_confi):
    nearest_idx = _nearest_index(queries, keys, label_confi)
    gather = _make_sc_gather(keys.shape[0], Q, D)
    return gather(nearest_idx, keys)


# speedup vs baseline: 1.0542x; 1.0542x over previous
import functools

import jax
import jax.numpy as jnp
from jax import lax
from jax.experimental import pallas as pl
from jax.experimental.pallas import tpu as pltpu
from jax.experimental.pallas import tpu_sc as plsc

Q = 1024
D = 128
KB = 4096
W = 4096
U = KB // W


def _argmin_body(q_ref, k_ref, lab_ref, kt_ref, labt_ref, idx_ref,
                 qn_ref, rmin_ref, rjdx_ref):
    j = pl.program_id(0)
    nblk = pl.num_programs(0)

    @pl.when(j == 0)
    def _init():
        q = q_ref[...]
        qn_ref[...] = (
            q / (jnp.sqrt(jnp.sum(q * q, axis=1, keepdims=True)) + 1e-12)
        ).astype(jnp.bfloat16)
        rmin_ref[...] = jnp.full_like(rmin_ref, jnp.inf)
        rjdx_ref[...] = jnp.zeros_like(rjdx_ref)

    k = k_ref[...]
    kn = (k / (jnp.sqrt(jnp.sum(k * k, axis=1, keepdims=True)) + 1e-12)
          ).astype(jnp.bfloat16)

    s = lax.dot_general(
        qn_ref[...], kn,
        dimension_numbers=(((1,), (1,)), ((), ())),
        preferred_element_type=jnp.float32,
    )
    lab = lab_ref[...].reshape(1, KB)
    pen = jnp.where(lab > 0, jnp.float32(1.0), jnp.float32(jnp.inf))
    dd = pen - s

    b = j * U
    vals = [dd[:, h * W:(h + 1) * W] for h in range(U)]
    idxs = [b + h for h in range(U)]
    while len(vals) > 1:
        nv, ni = [], []
        for p in range(0, len(vals) - 1, 2):
            cl = vals[p] <= vals[p + 1]
            nv.append(jnp.where(cl, vals[p], vals[p + 1]))
            ni.append(jnp.where(cl, idxs[p], idxs[p + 1]))
        if len(vals) % 2:
            nv.append(vals[-1])
            ni.append(idxs[-1])
        vals, idxs = nv, ni
    mloc, jloc = vals[0], idxs[0]

    upd = mloc < rmin_ref[...]
    rjdx_ref[...] = jnp.where(upd, jloc, rjdx_ref[...])
    rmin_ref[...] = jnp.where(upd, mloc, rmin_ref[...])

    @pl.when(j == nblk - 1)
    def _final():
        nmain = nblk * KB
        kt = kt_ref[...]
        knt = (kt / (jnp.sqrt(jnp.sum(kt * kt, axis=1, keepdims=True))
                     + 1e-12)).astype(jnp.bfloat16)
        st = lax.dot_general(
            qn_ref[...], knt,
            dimension_numbers=(((1,), (1,)), ((), ())),
            preferred_element_type=jnp.float32,
        )
        labt = labt_ref[...]
        pent = jnp.where(labt > 0, jnp.float32(1.0), jnp.float32(jnp.inf))
        ddt = pent - st
        tmin = jnp.min(ddt, axis=1, keepdims=True)
        targ = (jnp.argmin(ddt, axis=1).astype(jnp.int32).reshape(Q, 1)
                + nmain)

        rmin = rmin_ref[...]
        m = jnp.min(rmin, axis=1, keepdims=True)
        c = lax.broadcasted_iota(jnp.int32, (Q, W), 1)
        cand = jnp.where(rmin == m, rjdx_ref[...] * W + c,
                         jnp.int32(2**31 - 1))
        idx_main = jnp.min(cand, axis=1, keepdims=True)
        idx_ref[...] = jnp.where(tmin < m, targ, idx_main)


def _nearest_index(queries, keys, label_confi):
    K = keys.shape[0]
    nblk = K // KB
    nmain = nblk * KB
    kt = K - nmain
    lab = label_confi.astype(jnp.int32)
    lab3d = lab[:nmain].reshape(nblk, 1, KB)
    keys_tail = keys[nmain:]
    lab_tail = lab[nmain:].reshape(1, kt)
    idx = pl.pallas_call(
        _argmin_body,
        grid=(nblk,),
        in_specs=[
            pl.BlockSpec((Q, D), lambda j: (0, 0)),
            pl.BlockSpec((KB, D), lambda j: (j, 0)),
            pl.BlockSpec((1, 1, KB), lambda j: (j, 0, 0)),
            pl.BlockSpec((kt, D), lambda j: (0, 0)),
            pl.BlockSpec((1, kt), lambda j: (0, 0)),
        ],
        out_specs=pl.BlockSpec((Q, 1), lambda j: (0, 0)),
        out_shape=jax.ShapeDtypeStruct((Q, 1), jnp.int32),
        scratch_shapes=[
            pltpu.VMEM((Q, D), jnp.bfloat16),
            pltpu.VMEM((Q, W), jnp.float32),
            pltpu.VMEM((Q, W), jnp.int32),
        ],
    )(queries, keys, lab3d, keys_tail, lab_tail)
    return idx.reshape(Q)


def _make_sc_gather(V, B, Dm):
    NC, NS = 2, 16
    NW = NC * NS
    b_per_w = B // NW
    mesh = plsc.VectorSubcoreMesh(core_axis_name="c", subcore_axis_name="s")

    @functools.partial(
        pl.kernel,
        mesh=mesh,
        out_type=jax.ShapeDtypeStruct((B, Dm), jnp.float32),
        scratch_types=[
            pltpu.VMEM((b_per_w,), jnp.int32),
            pltpu.VMEM((b_per_w, Dm), jnp.float32),
            pltpu.SemaphoreType.DMA,
        ],
    )
    def gather_rows(idx_hbm, table_hbm, out_hbm, idx_v, rows_v, sem):
        wid = lax.axis_index("s") * NC + lax.axis_index("c")
        base = wid * b_per_w
        pltpu.sync_copy(idx_hbm.at[pl.ds(base, b_per_w)], idx_v)
        pltpu.async_copy(table_hbm.at[idx_v], rows_v, sem).wait()
        pltpu.sync_copy(rows_v, out_hbm.at[pl.ds(base, b_per_w)])

    return gather_rows


def kernel(queries, keys, label_confi):
    nearest_idx = _nearest_index(queries, keys, label_confi)
    gather = _make_sc_gather(keys.shape[0], Q, D)
    return gather(nearest_idx, keys)


# speedup vs baseline: 1.0749x; 1.0196x over previous
import functools

import jax
import jax.numpy as jnp
from jax import lax
from jax.experimental import pallas as pl
from jax.experimental.pallas import tpu as pltpu
from jax.experimental.pallas import tpu_sc as plsc

Q = 1024
D = 128
KB = 2048
W = 2048
U = KB // W


def _argmin_body(q_ref, k_ref, lab_ref, kt_ref, labt_ref, idx_ref,
                 qn_ref, rmin_ref, rjdx_ref):
    j = pl.program_id(0)
    nblk = pl.num_programs(0)

    @pl.when(j == 0)
    def _init():
        q = q_ref[...]
        qn_ref[...] = (
            q / (jnp.sqrt(jnp.sum(q * q, axis=1, keepdims=True)) + 1e-12)
        ).astype(jnp.bfloat16)
        rmin_ref[...] = jnp.full_like(rmin_ref, jnp.inf)
        rjdx_ref[...] = jnp.zeros_like(rjdx_ref)

    k = k_ref[...]
    kn = (k / (jnp.sqrt(jnp.sum(k * k, axis=1, keepdims=True)) + 1e-12)
          ).astype(jnp.bfloat16)

    s = lax.dot_general(
        qn_ref[...], kn,
        dimension_numbers=(((1,), (1,)), ((), ())),
        preferred_element_type=jnp.float32,
    )
    lab = lab_ref[...].reshape(1, KB)
    pen = jnp.where(lab > 0, jnp.float32(1.0), jnp.float32(jnp.inf))
    dd = pen - s

    b = j * U
    vals = [dd[:, h * W:(h + 1) * W] for h in range(U)]
    idxs = [b + h for h in range(U)]
    while len(vals) > 1:
        nv, ni = [], []
        for p in range(0, len(vals) - 1, 2):
            cl = vals[p] <= vals[p + 1]
            nv.append(jnp.where(cl, vals[p], vals[p + 1]))
            ni.append(jnp.where(cl, idxs[p], idxs[p + 1]))
        if len(vals) % 2:
            nv.append(vals[-1])
            ni.append(idxs[-1])
        vals, idxs = nv, ni
    mloc, jloc = vals[0], idxs[0]

    upd = mloc < rmin_ref[...]
    rjdx_ref[...] = jnp.where(upd, jloc, rjdx_ref[...])
    rmin_ref[...] = jnp.where(upd, mloc, rmin_ref[...])

    @pl.when(j == nblk - 1)
    def _final():
        nmain = nblk * KB
        kt = kt_ref[...]
        knt = (kt / (jnp.sqrt(jnp.sum(kt * kt, axis=1, keepdims=True))
                     + 1e-12)).astype(jnp.bfloat16)
        st = lax.dot_general(
            qn_ref[...], knt,
            dimension_numbers=(((1,), (1,)), ((), ())),
            preferred_element_type=jnp.float32,
        )
        labt = labt_ref[...]
        pent = jnp.where(labt > 0, jnp.float32(1.0), jnp.float32(jnp.inf))
        ddt = pent - st
        tmin = jnp.min(ddt, axis=1, keepdims=True)
        targ = (jnp.argmin(ddt, axis=1).astype(jnp.int32).reshape(Q, 1)
                + nmain)

        rmin = rmin_ref[...]
        m = jnp.min(rmin, axis=1, keepdims=True)
        c = lax.broadcasted_iota(jnp.int32, (Q, W), 1)
        cand = jnp.where(rmin == m, rjdx_ref[...] * W + c,
                         jnp.int32(2**31 - 1))
        idx_main = jnp.min(cand, axis=1, keepdims=True)
        idx_ref[...] = jnp.where(tmin < m, targ, idx_main)


def _nearest_index(queries, keys, label_confi):
    K = keys.shape[0]
    nblk = K // KB
    nmain = nblk * KB
    kt = K - nmain
    lab = label_confi.astype(jnp.int32)
    lab3d = lab[:nmain].reshape(nblk, 1, KB)
    keys_tail = keys[nmain:]
    lab_tail = lab[nmain:].reshape(1, kt)
    idx = pl.pallas_call(
        _argmin_body,
        grid=(nblk,),
        in_specs=[
            pl.BlockSpec((Q, D), lambda j: (0, 0)),
            pl.BlockSpec((KB, D), lambda j: (j, 0)),
            pl.BlockSpec((1, 1, KB), lambda j: (j, 0, 0)),
            pl.BlockSpec((kt, D), lambda j: (0, 0)),
            pl.BlockSpec((1, kt), lambda j: (0, 0)),
        ],
        out_specs=pl.BlockSpec((Q, 1), lambda j: (0, 0)),
        out_shape=jax.ShapeDtypeStruct((Q, 1), jnp.int32),
        scratch_shapes=[
            pltpu.VMEM((Q, D), jnp.bfloat16),
            pltpu.VMEM((Q, W), jnp.float32),
            pltpu.VMEM((Q, W), jnp.int32),
        ],
    )(queries, keys, lab3d, keys_tail, lab_tail)
    return idx.reshape(Q)


def _make_sc_gather(V, B, Dm):
    NC, NS = 2, 16
    NW = NC * NS
    b_per_w = B // NW
    mesh = plsc.VectorSubcoreMesh(core_axis_name="c", subcore_axis_name="s")

    @functools.partial(
        pl.kernel,
        mesh=mesh,
        out_type=jax.ShapeDtypeStruct((B, Dm), jnp.float32),
        scratch_types=[
            pltpu.VMEM((b_per_w,), jnp.int32),
            pltpu.VMEM((b_per_w, Dm), jnp.float32),
            pltpu.SemaphoreType.DMA,
        ],
    )
    def gather_rows(idx_hbm, table_hbm, out_hbm, idx_v, rows_v, sem):
        wid = lax.axis_index("s") * NC + lax.axis_index("c")
        base = wid * b_per_w
        pltpu.sync_copy(idx_hbm.at[pl.ds(base, b_per_w)], idx_v)
        pltpu.async_copy(table_hbm.at[idx_v], rows_v, sem).wait()
        pltpu.sync_copy(rows_v, out_hbm.at[pl.ds(base, b_per_w)])

    return gather_rows


def kernel(queries, keys, label_confi):
    nearest_idx = _nearest_index(queries, keys, label_confi)
    gather = _make_sc_gather(keys.shape[0], Q, D)
    return gather(nearest_idx, keys)
